# baseline (speedup 1.0000x reference)
import functools

import jax
import jax.numpy as jnp
from jax import lax
from jax.experimental import pallas as pl
from jax.experimental.pallas import tpu as pltpu
from jax.experimental.pallas import tpu_sc as plsc

N = 10000
E = 640000
FEAT_DIM = 116
NUM_TAB = 4
EMB_DIM = 16
D = 128
BN_INV = 1.0 / (1.0 + 1e-5) ** 0.5

NC = 2
NS = 16
NW = NC * NS
L = 16

NPAD = 10240
SPS = NPAD // NS
RPW = NPAD // NW

K = 128
CH = 160
EPAD = NW * CH * K
ZB = 64


def _emb_body(idx_hbm, feat_hbm, e0, e1, e2, e3, out_hbm,
              idx_v, t0, t1, t2, t3, feat_v, x_v):
    cid = lax.axis_index("c")
    sid = lax.axis_index("s")
    wid = sid * NC + cid
    base = wid * RPW
    pltpu.sync_copy(idx_hbm.at[wid], idx_v)
    for tab_hbm, tab_v in ((e0, t0), (e1, t1), (e2, t2), (e3, t3)):
        pltpu.sync_copy(tab_hbm, tab_v)

    @pl.when(wid < NW - 1)
    def _():
        pltpu.sync_copy(feat_hbm.at[pl.ds(base, RPW)], feat_v)

    @pl.when(wid == NW - 1)
    def _():
        pltpu.sync_copy(feat_hbm.at[pl.ds(N - 80, 80)], feat_v.at[pl.ds(0, 80)])

    lanes = lax.iota(jnp.int32, L)

    def block(b, carry):
        ivs = [idx_v[t, pl.ds(b * L, L)] for t in range(NUM_TAB)]
        rows = (b * L + lanes) * D
        for e in range(EMB_DIM):
            acc = plsc.load_gather(t0, [ivs[0] + e * 1000])
            acc = acc + plsc.load_gather(t1, [ivs[1] + e * 1000])
            acc = acc + plsc.load_gather(t2, [ivs[2] + e * 100])
            acc = acc + plsc.load_gather(t3, [ivs[3] + e * 100])
            plsc.store_scatter(x_v, [rows + e], acc)
        return carry

    lax.fori_loop(0, RPW // L, block, 0)

    def frow(r, carry):
        for j in range(7):
            x_v[pl.ds(r * D + EMB_DIM + j * L, L)] = feat_v[r, pl.ds(NUM_TAB + j * L, L)]
        return carry

    lax.fori_loop(0, RPW, frow, 0)
    pltpu.sync_copy(x_v, out_hbm.at[pl.ds(base * D, RPW * D)])


def _segsum_body(h_hbm, srcw, dstw, out_hbm, sidx, didx, gb0, gb1, zb, agg,
                 sg0, sg1, si0, si1, si2, si3):
    cid = lax.axis_index("c")
    sid = lax.axis_index("s")
    wid = sid * NC + cid
    si = (si0, si1, si2, si3)

    zeros = jnp.zeros((L,), jnp.float32)

    def zrow(r, carry):
        for j in range(D // L):
            zb[r, pl.ds(j * L, L)] = zeros
        return carry

    lax.fori_loop(0, ZB, zrow, 0)

    def zcopy(i, carry):
        pltpu.sync_copy(zb, agg.at[pl.ds(sid * SPS + i * ZB, ZB)])
        return carry

    lax.fori_loop(0, SPS // ZB, zcopy, 0)
    plsc.subcore_barrier()

    def fill_slot(r, ch):
        pltpu.async_copy(srcw.at[wid, ch], sidx.at[r], si[r])
        pltpu.async_copy(dstw.at[wid, ch], didx.at[r], si[r])

    def drain_slot(r):
        pltpu.make_async_copy(srcw.at[0, 0], sidx.at[r], si[r]).wait()
        pltpu.make_async_copy(srcw.at[0, 0], didx.at[r], si[r]).wait()

    for r in range(4):
        fill_slot(r, r)
    drain_slot(0)
    drain_slot(1)
    pltpu.async_copy(h_hbm.at[sidx.at[0]], gb0, sg0)
    pltpu.async_copy(h_hbm.at[sidx.at[1]], gb1, sg1)

    def step(i, carry):
        ch_base = 4 * i
        for u in range(4):
            ch = ch_base + u
            gb, sg = (gb0, sg0) if u % 2 == 0 else (gb1, sg1)
            r = u
            r2 = (u + 2) % 4
            pltpu.make_async_copy(h_hbm.at[pl.ds(0, K)], gb, sg).wait()
            pltpu.sync_copy(gb, agg.at[didx.at[r]], add=True)

            @pl.when(ch + 4 < CH)
            def _():
                fill_slot(r, ch + 4)

            @pl.when(ch + 2 < CH)
            def _():
                drain_slot(r2)
                pltpu.async_copy(h_hbm.at[sidx.at[r2]], gb, sg)

        return carry

    lax.fori_loop(0, CH // 4, step, 0)
    plsc.subcore_barrier()
    pltpu.sync_copy(
        agg.at[pl.ds(sid * SPS, SPS)],
        out_hbm.at[cid, pl.ds(sid * SPS, SPS)],
    )


def _make_sc_kernels(interpret=False):
    mesh = plsc.VectorSubcoreMesh(core_axis_name="c", subcore_axis_name="s",
                                  num_cores=NC, num_subcores=NS)
    emb_kernel = pl.kernel(
        _emb_body,
        out_type=jax.ShapeDtypeStruct((NPAD * D,), jnp.float32),
        mesh=mesh,
        scratch_types=[
            pltpu.VMEM((NUM_TAB, RPW), jnp.int32),
            pltpu.VMEM((EMB_DIM * 1000,), jnp.float32),
            pltpu.VMEM((EMB_DIM * 1000,), jnp.float32),
            pltpu.VMEM((EMB_DIM * 100,), jnp.float32),
            pltpu.VMEM((EMB_DIM * 100,), jnp.float32),
            pltpu.VMEM((RPW, FEAT_DIM), jnp.float32),
            pltpu.VMEM((RPW * D,), jnp.float32),
        ],
        compiler_params=pltpu.CompilerParams(needs_layout_passes=False),
        interpret=interpret,
    )
    segsum_kernel = pl.kernel(
        _segsum_body,
        out_type=jax.ShapeDtypeStruct((NC, NPAD, D), jnp.float32),
        mesh=mesh,
        scratch_types=[
            pltpu.VMEM((4, K), jnp.int32),
            pltpu.VMEM((4, K), jnp.int32),
            pltpu.VMEM((K, D), jnp.float32),
            pltpu.VMEM((K, D), jnp.float32),
            pltpu.VMEM((ZB, D), jnp.float32),
            pltpu.VMEM_SHARED((NPAD, D), jnp.float32),
            pltpu.SemaphoreType.DMA,
            pltpu.SemaphoreType.DMA,
            pltpu.SemaphoreType.DMA,
            pltpu.SemaphoreType.DMA,
            pltpu.SemaphoreType.DMA,
            pltpu.SemaphoreType.DMA,
        ],
        interpret=interpret,
    )
    return emb_kernel, segsum_kernel


_emb_kernel, _segsum_kernel = _make_sc_kernels()


def _mlp_body(relu_out, jk, *refs):
    if jk:
        h_ref, p_ref, w1_ref, b1_ref, w2_ref, b2_ref, g_ref, bt_ref, x_ref, out_ref = refs
    else:
        h_ref, p_ref, w1_ref, b1_ref, w2_ref, b2_ref, g_ref, bt_ref, out_ref = refs
    t = h_ref[...] + p_ref[0] + p_ref[1]
    hid = jnp.dot(t, w1_ref[...], preferred_element_type=jnp.float32,
                  precision=lax.Precision.HIGHEST)
    hid = jnp.maximum(hid + b1_ref[...], 0.0)
    o = jnp.dot(hid, w2_ref[...], preferred_element_type=jnp.float32,
                precision=lax.Precision.HIGHEST)
    o = (o + b2_ref[...]) * (g_ref[...] * BN_INV) + bt_ref[...]
    if relu_out:
        o = jnp.maximum(o, 0.0)
    if jk:
        o = o + x_ref[...] + h_ref[...]
    out_ref[...] = o


def _mlp_layer(h, parts, w1, b1, w2, b2, gamma, beta, relu_out, x=None):
    R = 1000
    grid = (N // R,)
    jk = x is not None
    in_specs = [
        pl.BlockSpec((R, D), lambda i: (i, 0)),
        pl.BlockSpec((NC, R, D), lambda i: (0, i, 0)),
        pl.BlockSpec((D, 2 * D), lambda i: (0, 0)),
        pl.BlockSpec((1, 2 * D), lambda i: (0, 0)),
        pl.BlockSpec((2 * D, D), lambda i: (0, 0)),
        pl.BlockSpec((1, D), lambda i: (0, 0)),
        pl.BlockSpec((1, D), lambda i: (0, 0)),
        pl.BlockSpec((1, D), lambda i: (0, 0)),
    ]
    args = [h, parts, w1, b1.reshape(1, -1), w2, b2.reshape(1, -1),
            gamma.reshape(1, -1), beta.reshape(1, -1)]
    if jk:
        in_specs.append(pl.BlockSpec((R, D), lambda i: (i, 0)))
        args.append(x)
    return pl.pallas_call(
        functools.partial(_mlp_body, relu_out, jk),
        grid=grid,
        in_specs=in_specs,
        out_specs=pl.BlockSpec((R, D), lambda i: (i, 0)),
        out_shape=jax.ShapeDtypeStruct((N, D), jnp.float32),
    )(*args)


def _prep_inputs(feat, edge_index):
    idx = feat[:, :NUM_TAB].astype(jnp.int32)
    idx = jnp.pad(idx, ((0, NPAD - N), (0, 0)))
    idx = idx.T.reshape(NUM_TAB, NW, RPW).transpose(1, 0, 2)

    pad_e = EPAD - E
    ar = jnp.arange(pad_e, dtype=jnp.int32)
    pad_src = (ar * 97) % N
    pad_dst = N + ar % (NPAD - N)
    srcw = jnp.concatenate([edge_index[0], pad_src]).reshape(NW, CH, K)
    dstw = jnp.concatenate([edge_index[1], pad_dst]).reshape(NW, CH, K)
    return idx, srcw, dstw


def kernel(feat, edge_index, emb0, emb1, emb2, emb3, W1_0, b1_0, W2_0, b2_0,
           W1_1, b1_1, W2_1, b2_1, gamma0, beta0, gamma1, beta1):
    idx, srcw, dstw = _prep_inputs(feat, edge_index)

    x = _emb_kernel(idx, feat, emb0.T.reshape(-1), emb1.T.reshape(-1),
                    emb2.T.reshape(-1), emb3.T.reshape(-1)).reshape(NPAD, D)

    parts0 = _segsum_kernel(x, srcw, dstw)
    h1 = _mlp_layer(x, parts0, W1_0, b1_0, W2_0, b2_0, gamma0, beta0,
                    relu_out=True)
    parts1 = _segsum_kernel(h1, srcw, dstw)
    out = _mlp_layer(h1, parts1, W1_1, b1_1, W2_1, b2_1, gamma1, beta1,
                     relu_out=False, x=x)
    return out


# speedup vs baseline: 1.0087x; 1.0087x over previous
import functools

import jax
import jax.numpy as jnp
from jax import lax
from jax.experimental import pallas as pl
from jax.experimental.pallas import tpu as pltpu
from jax.experimental.pallas import tpu_sc as plsc

N = 10000
E = 640000
FEAT_DIM = 116
NUM_TAB = 4
EMB_DIM = 16
D = 128
BN_INV = 1.0 / (1.0 + 1e-5) ** 0.5

NC = 2
NS = 16
NW = NC * NS
L = 16

NPAD = 10240
SPS = NPAD // NS
RPW = NPAD // NW

K = 128
CH = 160
EPAD = NW * CH * K
ZB = 64


def _emb_body(idx_hbm, feat_hbm, e0, e1, e2, e3, out_hbm,
              idx_v, t0, t1, t2, t3, feat_v, x_v):
    cid = lax.axis_index("c")
    sid = lax.axis_index("s")
    wid = sid * NC + cid
    base = wid * RPW
    pltpu.sync_copy(idx_hbm.at[wid], idx_v)
    for tab_hbm, tab_v in ((e0, t0), (e1, t1), (e2, t2), (e3, t3)):
        pltpu.sync_copy(tab_hbm, tab_v)

    @pl.when(wid < NW - 1)
    def _():
        pltpu.sync_copy(feat_hbm.at[pl.ds(base, RPW)], feat_v)

    @pl.when(wid == NW - 1)
    def _():
        pltpu.sync_copy(feat_hbm.at[pl.ds(N - 80, 80)], feat_v.at[pl.ds(0, 80)])

    lanes = lax.iota(jnp.int32, L)

    def block(b, carry):
        ivs = [idx_v[t, pl.ds(b * L, L)] for t in range(NUM_TAB)]
        rows = (b * L + lanes) * D
        for e in range(EMB_DIM):
            acc = plsc.load_gather(t0, [ivs[0] + e * 1000])
            acc = acc + plsc.load_gather(t1, [ivs[1] + e * 1000])
            acc = acc + plsc.load_gather(t2, [ivs[2] + e * 100])
            acc = acc + plsc.load_gather(t3, [ivs[3] + e * 100])
            plsc.store_scatter(x_v, [rows + e], acc)
        return carry

    lax.fori_loop(0, RPW // L, block, 0)

    def frow(r, carry):
        for j in range(7):
            x_v[pl.ds(r * D + EMB_DIM + j * L, L)] = feat_v[r, pl.ds(NUM_TAB + j * L, L)]
        return carry

    lax.fori_loop(0, RPW, frow, 0)
    pltpu.sync_copy(x_v, out_hbm.at[pl.ds(base * D, RPW * D)])


def _segsum_body(h_hbm, srcw, dstw, out_hbm, sidx, didx, gb0, gb1, zb, agg,
                 sg0, sg1, si0, si1, si2, si3):
    cid = lax.axis_index("c")
    sid = lax.axis_index("s")
    wid = sid * NC + cid
    si = (si0, si1, si2, si3)

    def fill_slot(r, ch):
        pltpu.async_copy(srcw.at[wid, ch], sidx.at[r], si[r])
        pltpu.async_copy(dstw.at[wid, ch], didx.at[r], si[r])

    def drain_slot(r):
        pltpu.make_async_copy(srcw.at[0, 0], sidx.at[r], si[r]).wait()
        pltpu.make_async_copy(srcw.at[0, 0], didx.at[r], si[r]).wait()

    for r in range(4):
        fill_slot(r, r)
    drain_slot(0)
    drain_slot(1)
    pltpu.async_copy(h_hbm.at[sidx.at[0]], gb0, sg0)
    pltpu.async_copy(h_hbm.at[sidx.at[1]], gb1, sg1)

    zeros = jnp.zeros((L,), jnp.float32)

    def zrow(r, carry):
        for j in range(D // L):
            zb[r, pl.ds(j * L, L)] = zeros
        return carry

    lax.fori_loop(0, ZB, zrow, 0)

    def zcopy(i, carry):
        pltpu.sync_copy(zb, agg.at[pl.ds(sid * SPS + i * ZB, ZB)])
        return carry

    lax.fori_loop(0, SPS // ZB, zcopy, 0)
    plsc.subcore_barrier()

    def step(i, carry):
        ch_base = 4 * i
        for u in range(4):
            ch = ch_base + u
            gb, sg = (gb0, sg0) if u % 2 == 0 else (gb1, sg1)
            r = u
            r2 = (u + 2) % 4
            pltpu.make_async_copy(h_hbm.at[pl.ds(0, K)], gb, sg).wait()
            pltpu.sync_copy(gb, agg.at[didx.at[r]], add=True)

            @pl.when(ch + 4 < CH)
            def _():
                fill_slot(r, ch + 4)

            @pl.when(ch + 2 < CH)
            def _():
                drain_slot(r2)
                pltpu.async_copy(h_hbm.at[sidx.at[r2]], gb, sg)

        return carry

    lax.fori_loop(0, CH // 4, step, 0)
    plsc.subcore_barrier()
    pltpu.sync_copy(
        agg.at[pl.ds(sid * SPS, SPS)],
        out_hbm.at[cid, pl.ds(sid * SPS, SPS)],
    )


def _make_sc_kernels(interpret=False):
    mesh = plsc.VectorSubcoreMesh(core_axis_name="c", subcore_axis_name="s",
                                  num_cores=NC, num_subcores=NS)
    emb_kernel = pl.kernel(
        _emb_body,
        out_type=jax.ShapeDtypeStruct((NPAD * D,), jnp.float32),
        mesh=mesh,
        scratch_types=[
            pltpu.VMEM((NUM_TAB, RPW), jnp.int32),
            pltpu.VMEM((EMB_DIM * 1000,), jnp.float32),
            pltpu.VMEM((EMB_DIM * 1000,), jnp.float32),
            pltpu.VMEM((EMB_DIM * 100,), jnp.float32),
            pltpu.VMEM((EMB_DIM * 100,), jnp.float32),
            pltpu.VMEM((RPW, FEAT_DIM), jnp.float32),
            pltpu.VMEM((RPW * D,), jnp.float32),
        ],
        compiler_params=pltpu.CompilerParams(needs_layout_passes=False),
        interpret=interpret,
    )
    segsum_kernel = pl.kernel(
        _segsum_body,
        out_type=jax.ShapeDtypeStruct((NC, NPAD, D), jnp.float32),
        mesh=mesh,
        scratch_types=[
            pltpu.VMEM((4, K), jnp.int32),
            pltpu.VMEM((4, K), jnp.int32),
            pltpu.VMEM((K, D), jnp.float32),
            pltpu.VMEM((K, D), jnp.float32),
            pltpu.VMEM((ZB, D), jnp.float32),
            pltpu.VMEM_SHARED((NPAD, D), jnp.float32),
            pltpu.SemaphoreType.DMA,
            pltpu.SemaphoreType.DMA,
            pltpu.SemaphoreType.DMA,
            pltpu.SemaphoreType.DMA,
            pltpu.SemaphoreType.DMA,
            pltpu.SemaphoreType.DMA,
        ],
        interpret=interpret,
    )
    return emb_kernel, segsum_kernel


_emb_kernel, _segsum_kernel = _make_sc_kernels()


def _mlp_body(relu_out, jk, *refs):
    if jk:
        h_ref, p_ref, w1_ref, b1_ref, w2_ref, b2_ref, g_ref, bt_ref, x_ref, out_ref = refs
    else:
        h_ref, p_ref, w1_ref, b1_ref, w2_ref, b2_ref, g_ref, bt_ref, out_ref = refs
    t = h_ref[...] + p_ref[0] + p_ref[1]
    hid = jnp.dot(t, w1_ref[...], preferred_element_type=jnp.float32,
                  precision=lax.Precision.HIGHEST)
    hid = jnp.maximum(hid + b1_ref[...], 0.0)
    o = jnp.dot(hid, w2_ref[...], preferred_element_type=jnp.float32,
                precision=lax.Precision.HIGHEST)
    o = (o + b2_ref[...]) * (g_ref[...] * BN_INV) + bt_ref[...]
    if relu_out:
        o = jnp.maximum(o, 0.0)
    if jk:
        o = o + x_ref[...] + h_ref[...]
    out_ref[...] = o


def _mlp_layer(h, parts, w1, b1, w2, b2, gamma, beta, relu_out, x=None):
    R = 1000
    grid = (N // R,)
    jk = x is not None
    in_specs = [
        pl.BlockSpec((R, D), lambda i: (i, 0)),
        pl.BlockSpec((NC, R, D), lambda i: (0, i, 0)),
        pl.BlockSpec((D, 2 * D), lambda i: (0, 0)),
        pl.BlockSpec((1, 2 * D), lambda i: (0, 0)),
        pl.BlockSpec((2 * D, D), lambda i: (0, 0)),
        pl.BlockSpec((1, D), lambda i: (0, 0)),
        pl.BlockSpec((1, D), lambda i: (0, 0)),
        pl.BlockSpec((1, D), lambda i: (0, 0)),
    ]
    args = [h, parts, w1, b1.reshape(1, -1), w2, b2.reshape(1, -1),
            gamma.reshape(1, -1), beta.reshape(1, -1)]
    if jk:
        in_specs.append(pl.BlockSpec((R, D), lambda i: (i, 0)))
        args.append(x)
    return pl.pallas_call(
        functools.partial(_mlp_body, relu_out, jk),
        grid=grid,
        in_specs=in_specs,
        out_specs=pl.BlockSpec((R, D), lambda i: (i, 0)),
        out_shape=jax.ShapeDtypeStruct((N, D), jnp.float32),
    )(*args)


def _prep_inputs(feat, edge_index):
    idx = feat[:, :NUM_TAB].astype(jnp.int32)
    idx = jnp.pad(idx, ((0, NPAD - N), (0, 0)))
    idx = idx.T.reshape(NUM_TAB, NW, RPW).transpose(1, 0, 2)

    pad_e = EPAD - E
    ar = jnp.arange(pad_e, dtype=jnp.int32)
    pad_src = (ar * 97) % N
    pad_dst = N + ar % (NPAD - N)
    srcw = jnp.concatenate([edge_index[0], pad_src]).reshape(NW, CH, K)
    dstw = jnp.concatenate([edge_index[1], pad_dst]).reshape(NW, CH, K)
    return idx, srcw, dstw


def kernel(feat, edge_index, emb0, emb1, emb2, emb3, W1_0, b1_0, W2_0, b2_0,
           W1_1, b1_1, W2_1, b2_1, gamma0, beta0, gamma1, beta1):
    idx, srcw, dstw = _prep_inputs(feat, edge_index)

    x = _emb_kernel(idx, feat, emb0.T.reshape(-1), emb1.T.reshape(-1),
                    emb2.T.reshape(-1), emb3.T.reshape(-1)).reshape(NPAD, D)

    parts0 = _segsum_kernel(x, srcw, dstw)
    h1 = _mlp_layer(x, parts0, W1_0, b1_0, W2_0, b2_0, gamma0, beta0,
                    relu_out=True)
    parts1 = _segsum_kernel(h1, srcw, dstw)
    out = _mlp_layer(h1, parts1, W1_1, b1_1, W2_1, b2_1, gamma1, beta1,
                     relu_out=False, x=x)
    return out
